# baseline (speedup 1.0000x reference)
import functools

import jax
import jax.numpy as jnp
from jax import lax
from jax.experimental import pallas as pl
from jax.experimental.pallas import tpu as pltpu
from jax.experimental.pallas import tpu_sc as plsc

_B, _C, _H, _W, _N = 4, 768, 224, 224, 4096
_HW = _H * _W
_BC = _B * _C
_NW = 32
_L = 16
_PAIRS = _BC // _NW
_NCHUNK = _N // _L


def _sc_body(o1_hbm, o2_hbm, r1_hbm, c1_hbm, r2_hbm, c2_hbm, out_hbm,
             r_v, c_v, lin1_v, lin2_v, v1a_v, v1b_v, v2a_v, v2b_v, acc_v,
             sem1a, sem1b, sem2a, sem2b):
    cid = lax.axis_index("c")
    sid = lax.axis_index("s")
    wid = sid * 2 + cid
    b = wid // 8
    p0 = wid * _PAIRS

    pltpu.sync_copy(r1_hbm.at[b], r_v)
    pltpu.sync_copy(c1_hbm.at[b], c_v)

    def lin1_body(k, u):
        s = k * _L
        lin1_v[pl.ds(s, _L)] = r_v[pl.ds(s, _L)] * _W + c_v[pl.ds(s, _L)]
        return u

    lax.fori_loop(0, _NCHUNK, lin1_body, 0)
    pltpu.sync_copy(r2_hbm.at[b], r_v)
    pltpu.sync_copy(c2_hbm.at[b], c_v)

    def lin2_body(k, u):
        s = k * _L
        lin2_v[pl.ds(s, _L)] = r_v[pl.ds(s, _L)] * _W + c_v[pl.ds(s, _L)]
        return u

    lax.fori_loop(0, _NCHUNK, lin2_body, 0)

    acc_v[...] = jnp.zeros((_L,), jnp.float32)

    def issue(p, v1_ref, v2_ref, s1, s2):
        pltpu.async_copy(o1_hbm.at[p].at[lin1_v], v1_ref, s1)
        pltpu.async_copy(o2_hbm.at[p].at[lin2_v], v2_ref, s2)

    def drain_acc(p, v1_ref, v2_ref, s1, s2):
        pltpu.make_async_copy(o1_hbm.at[p].at[lin1_v], v1_ref, s1).wait()
        pltpu.make_async_copy(o2_hbm.at[p].at[lin2_v], v2_ref, s2).wait()

        lane = lax.iota(jnp.int32, _L)
        zero = jnp.zeros((_L,), jnp.int32)

        def inner(k, u):
            s = k * _L
            d = v1_ref[pl.ds(s, _L)] - v2_ref[pl.ds(s, _L)]
            acc_v[...] = acc_v[...] + d * d
            return u

        lax.fori_loop(0, _NCHUNK, inner, 0)

    issue(p0, v1a_v, v2a_v, sem1a, sem2a)

    def pair_body(i, u):
        j = 2 * i
        issue(p0 + j + 1, v1b_v, v2b_v, sem1b, sem2b)
        drain_acc(p0 + j, v1a_v, v2a_v, sem1a, sem2a)
        issue(p0 + j + 2, v1a_v, v2a_v, sem1a, sem2a)
        drain_acc(p0 + j + 1, v1b_v, v2b_v, sem1b, sem2b)
        return u

    lax.fori_loop(0, (_PAIRS - 2) // 2, pair_body, 0)
    j_last = _PAIRS - 2
    issue(p0 + j_last + 1, v1b_v, v2b_v, sem1b, sem2b)
    drain_acc(p0 + j_last, v1a_v, v2a_v, sem1a, sem2a)
    drain_acc(p0 + j_last + 1, v1b_v, v2b_v, sem1b, sem2b)

    pltpu.sync_copy(acc_v, out_hbm.at[wid])


@jax.jit
def _sc_loss(o1, o2, r1, c1, r2, c2):
    mesh = plsc.VectorSubcoreMesh(core_axis_name="c", subcore_axis_name="s")
    parts = pl.kernel(
        _sc_body,
        out_type=jax.ShapeDtypeStruct((_NW, _L), jnp.float32),
        mesh=mesh,
        compiler_params=pltpu.CompilerParams(
            needs_layout_passes=False, use_tc_tiling_on_sc=False),
        scratch_types=[
            pltpu.VMEM((_N,), jnp.int32),
            pltpu.VMEM((_N,), jnp.int32),
            pltpu.VMEM((_N,), jnp.int32),
            pltpu.VMEM((_N,), jnp.int32),
            pltpu.VMEM((_N,), jnp.float32),
            pltpu.VMEM((_N,), jnp.float32),
            pltpu.VMEM((_N,), jnp.float32),
            pltpu.VMEM((_N,), jnp.float32),
            pltpu.VMEM((_L,), jnp.float32),
            pltpu.SemaphoreType.DMA,
            pltpu.SemaphoreType.DMA,
            pltpu.SemaphoreType.DMA,
            pltpu.SemaphoreType.DMA,
        ],
    )(o1, o2, r1, c1, r2, c2)
    return jnp.sum(parts) * (1.0 / (_B * _N))


def kernel(out_1, out_2, match_1, match_2, nonmatch_2):
    del nonmatch_2
    o1 = out_1.reshape(_BC, _HW)
    o2 = out_2.reshape(_BC, _HW)
    r1 = match_1[:, :, 0]
    c1 = match_1[:, :, 1]
    r2 = match_2[:, :, 0]
    c2 = match_2[:, :, 1]
    return _sc_loss(o1, o2, r1, c1, r2, c2)


# speedup vs baseline: 1.6158x; 1.6158x over previous
import functools

import jax
import jax.numpy as jnp
from jax import lax
from jax.experimental import pallas as pl
from jax.experimental.pallas import tpu as pltpu
from jax.experimental.pallas import tpu_sc as plsc

_B, _C, _H, _W, _N = 4, 768, 224, 224, 4096
_NW = 32
_L = 16
_PAIRS = _B * _C // _NW
_NCHUNK = _N // _L


def _sc_body(o1_hbm, o2_hbm, mm_hbm, out_hbm,
             plane_a, plane_b, vals1_v, vals2_v, mm_v, acc_v,
             sem_a, sem_b):
    cid = lax.axis_index("c")
    sid = lax.axis_index("s")
    wid = sid * 2 + cid
    b = wid // 8
    ch0 = (wid % 8) * _PAIRS

    pltpu.sync_copy(mm_hbm.at[b], mm_v)

    acc_v[...] = jnp.zeros((_L,), jnp.float32)

    def start_a(ch):
        pltpu.async_copy(o1_hbm.at[b, ch], plane_a, sem_a)

    def start_b(ch):
        pltpu.async_copy(o2_hbm.at[b, ch], plane_b, sem_b)

    def wait_a(ch):
        pltpu.make_async_copy(o1_hbm.at[b, ch], plane_a, sem_a).wait()

    def wait_b(ch):
        pltpu.make_async_copy(o2_hbm.at[b, ch], plane_b, sem_b).wait()

    def gather1(_):
        def body(k, u):
            s = k * _L
            m = mm_v[pl.ds(s, _L)]
            vals1_v[pl.ds(s, _L)] = plsc.load_gather(
                plane_a, [(m >> 24) & 0xFF, (m >> 16) & 0xFF])
            return u

        lax.fori_loop(0, _NCHUNK, body, 0, unroll=4)

    def gather2(_):
        def body(k, u):
            s = k * _L
            m = mm_v[pl.ds(s, _L)]
            vals2_v[pl.ds(s, _L)] = plsc.load_gather(
                plane_b, [(m >> 8) & 0xFF, m & 0xFF])
            return u

        lax.fori_loop(0, _NCHUNK, body, 0, unroll=4)

    def accumulate(_):
        def body(k, u):
            s = k * _L
            d = vals1_v[pl.ds(s, _L)] - vals2_v[pl.ds(s, _L)]
            acc_v[...] = acc_v[...] + d * d
            return u

        lax.fori_loop(0, _NCHUNK, body, 0, unroll=4)

    start_a(ch0)
    start_b(ch0)

    def plane_body(j, u):
        ch = ch0 + j
        wait_a(ch)
        gather1(None)
        start_a(ch + 1)
        wait_b(ch)
        gather2(None)
        start_b(ch + 1)
        accumulate(None)
        return u

    lax.fori_loop(0, _PAIRS - 1, plane_body, 0)
    ch_last = ch0 + _PAIRS - 1
    wait_a(ch_last)
    gather1(None)
    wait_b(ch_last)
    gather2(None)
    accumulate(None)

    pltpu.sync_copy(acc_v, out_hbm.at[wid])


@jax.jit
def _sc_loss(o1, o2, mm):
    mesh = plsc.VectorSubcoreMesh(core_axis_name="c", subcore_axis_name="s")
    parts = pl.kernel(
        _sc_body,
        out_type=jax.ShapeDtypeStruct((_NW, _L), jnp.float32),
        mesh=mesh,
        compiler_params=pltpu.CompilerParams(needs_layout_passes=False),
        scratch_types=[
            pltpu.VMEM((_H, _W), jnp.float32),
            pltpu.VMEM((_H, _W), jnp.float32),
            pltpu.VMEM((_N,), jnp.float32),
            pltpu.VMEM((_N,), jnp.float32),
            pltpu.VMEM((_N,), jnp.int32),
            pltpu.VMEM((_L,), jnp.float32),
            pltpu.SemaphoreType.DMA,
            pltpu.SemaphoreType.DMA,
        ],
    )(o1, o2, mm)
    return jnp.sum(parts) * (1.0 / (_B * _N))


def kernel(out_1, out_2, match_1, match_2, nonmatch_2):
    del nonmatch_2
    mm = ((match_1[:, :, 0] << 24) | (match_1[:, :, 1] << 16)
          | (match_2[:, :, 0] << 8) | match_2[:, :, 1])
    return _sc_loss(out_1, out_2, mm)


# speedup vs baseline: 14.9151x; 9.2306x over previous
import functools

import jax
import jax.numpy as jnp
from jax import lax
from jax.experimental import pallas as pl
from jax.experimental.pallas import tpu as pltpu
from jax.experimental.pallas import tpu_sc as plsc

_B, _C, _H, _W, _N = 4, 768, 224, 224, 4096
_NW = 32
_L = 16
_PTS = _N // 8
_KC = _C // 128
_CP = 32
_NCH = _PTS // _CP
_RC = _CP * _KC
_V = _B * _H * (_W // 8) * _KC * 8


def _sc_body(o1_hbm, o2_hbm, m1_hbm, m2_hbm, out_hbm,
             m1_v, m2_v, base_v, idx1_v, idx2_v,
             v1a, v1b, v2a, v2b, acc_v,
             sem1, sem2):
    cid = lax.axis_index("c")
    sid = lax.axis_index("s")
    wid = sid * 2 + cid
    b = wid // 8
    n0 = (wid % 8) * _PTS

    pltpu.sync_copy(m1_hbm.at[b, pl.ds(n0, _PTS)], m1_v)
    pltpu.sync_copy(m2_hbm.at[b, pl.ds(n0, _PTS)], m2_v)

    bh = b * _H

    def build_idx(m_v, idx_v):
        def base_body(t, u):
            s = t * _L
            m = m_v[pl.ds(s, _L)]
            r = m >> 16
            c = m & 0xFFFF
            base_v[pl.ds(s, _L)] = ((bh + r) * (_W // 8) + (c >> 3)) \
                * (_KC * 8) + (c & 7)
            return u

        lax.fori_loop(0, _PTS // _L, base_body, 0, unroll=4)
        for k in range(_KC):
            def k_body(t, u, k=k):
                s = t * _L
                ci = t >> 1
                off = ci * _RC + k * _CP + (t & 1) * _L
                idx_v[pl.ds(off, _L)] = base_v[pl.ds(s, _L)] + (k * 8)
                return u

            lax.fori_loop(0, _PTS // _L, k_body, 0, unroll=4)

    build_idx(m1_v, idx1_v)
    build_idx(m2_v, idx2_v)

    acc_v[...] = jnp.zeros((_L,), jnp.float32)

    def issue(ci, v1_ref, v2_ref):
        s = ci * _RC
        pltpu.async_copy(o1_hbm.at[idx1_v.at[pl.ds(s, _RC)]], v1_ref, sem1)
        pltpu.async_copy(o2_hbm.at[idx2_v.at[pl.ds(s, _RC)]], v2_ref, sem2)

    lane = lax.iota(jnp.int32, _L)

    def drain_acc(ci, v1_ref, v2_ref):
        s = ci * _RC
        pltpu.make_async_copy(
            o1_hbm.at[idx1_v.at[pl.ds(s, _RC)]], v1_ref, sem1).wait()
        pltpu.make_async_copy(
            o2_hbm.at[idx2_v.at[pl.ds(s, _RC)]], v2_ref, sem2).wait()

        def body(t, u):
            jv = jnp.broadcast_to(t >> 3, (_L,)).astype(jnp.int32)
            wv = ((t & 7) << 4) + lane
            d = plsc.load_gather(v1_ref, [jv, wv]) \
                - plsc.load_gather(v2_ref, [jv, wv])
            acc_v[...] = acc_v[...] + d * d
            return u

        lax.fori_loop(0, _RC * 8, body, 0, unroll=4)

    issue(0, v1a, v2a)

    def pair_body(i, u):
        j = 2 * i
        issue(j + 1, v1b, v2b)
        drain_acc(j, v1a, v2a)
        issue(j + 2, v1a, v2a)
        drain_acc(j + 1, v1b, v2b)
        return u

    lax.fori_loop(0, (_NCH - 2) // 2, pair_body, 0)
    issue(_NCH - 1, v1b, v2b)
    drain_acc(_NCH - 2, v1a, v2a)
    drain_acc(_NCH - 1, v1b, v2b)

    pltpu.sync_copy(acc_v, out_hbm.at[wid])


@jax.jit
def _sc_loss(o1, o2, m1, m2):
    mesh = plsc.VectorSubcoreMesh(core_axis_name="c", subcore_axis_name="s")
    parts = pl.kernel(
        _sc_body,
        out_type=jax.ShapeDtypeStruct((_NW, _L), jnp.float32),
        mesh=mesh,
        compiler_params=pltpu.CompilerParams(needs_layout_passes=False),
        scratch_types=[
            pltpu.VMEM((_PTS,), jnp.int32),
            pltpu.VMEM((_PTS,), jnp.int32),
            pltpu.VMEM((_PTS,), jnp.int32),
            pltpu.VMEM((_PTS * _KC,), jnp.int32),
            pltpu.VMEM((_PTS * _KC,), jnp.int32),
            pltpu.VMEM((_RC, 128), jnp.float32),
            pltpu.VMEM((_RC, 128), jnp.float32),
            pltpu.VMEM((_RC, 128), jnp.float32),
            pltpu.VMEM((_RC, 128), jnp.float32),
            pltpu.VMEM((_L,), jnp.float32),
            pltpu.SemaphoreType.DMA,
            pltpu.SemaphoreType.DMA,
        ],
    )(o1, o2, m1, m2)
    return jnp.sum(parts) * (1.0 / (_B * _N))


def kernel(out_1, out_2, match_1, match_2, nonmatch_2):
    del nonmatch_2

    def rowview(x):
        y = x.transpose(0, 2, 3, 1)
        y = y.reshape(_B, _H, _W // 8, 8, _KC, 128)
        y = y.transpose(0, 1, 2, 4, 3, 5)
        return y.reshape(_V, 128)

    m1 = (match_1[:, :, 0] << 16) | match_1[:, :, 1]
    m2 = (match_2[:, :, 0] << 16) | match_2[:, :, 1]
    return _sc_loss(rowview(out_1), rowview(out_2), m1, m2)


# speedup vs baseline: 49.7418x; 3.3350x over previous
import functools

import jax
import jax.numpy as jnp
from jax import lax
from jax.experimental import pallas as pl
from jax.experimental.pallas import tpu as pltpu
from jax.experimental.pallas import tpu_sc as plsc

_B, _C, _H, _W, _N = 4, 768, 224, 224, 4096
_NW = 32
_L = 16
_PTS = _N // 8
_KC = _C // 128
_CP = 32
_NCH = _PTS // _CP
_RC = _CP * _KC
_V = _B * _H * (_W // 8) * _KC * 8


def _sc_body(o1_hbm, o2_hbm, m1_hbm, m2_hbm, out_hbm,
             m1_v, m2_v, base_v, idx1_v, idx2_v,
             v1a, v1b, v2a, v2b, acc_v,
             sem1, sem2):
    cid = lax.axis_index("c")
    sid = lax.axis_index("s")
    wid = sid * 2 + cid
    b = wid // 8
    n0 = (wid % 8) * _PTS

    pltpu.sync_copy(m1_hbm.at[b, pl.ds(n0, _PTS)], m1_v)
    pltpu.sync_copy(m2_hbm.at[b, pl.ds(n0, _PTS)], m2_v)

    bh = b * _H

    def build_idx(m_v, idx_v):
        def base_body(t, u):
            s = t * _L
            m = m_v[pl.ds(s, _L)]
            r = m >> 16
            c = m & 0xFFFF
            base_v[pl.ds(s, _L)] = ((bh + r) * (_W // 8) + (c >> 3)) \
                * (_KC * 8) + (c & 7)
            return u

        lax.fori_loop(0, _PTS // _L, base_body, 0, unroll=4)
        for k in range(_KC):
            def k_body(t, u, k=k):
                s = t * _L
                ci = t >> 1
                off = ci * _RC + k * _CP + (t & 1) * _L
                idx_v[pl.ds(off, _L)] = base_v[pl.ds(s, _L)] + (k * 8)
                return u

            lax.fori_loop(0, _PTS // _L, k_body, 0, unroll=4)

    build_idx(m1_v, idx1_v)
    build_idx(m2_v, idx2_v)

    acc_v[...] = jnp.zeros((_L,), jnp.float32)

    def issue(ci, v1_ref, v2_ref):
        s = ci * _RC
        pltpu.async_copy(o1_hbm.at[idx1_v.at[pl.ds(s, _RC)]], v1_ref, sem1)
        pltpu.async_copy(o2_hbm.at[idx2_v.at[pl.ds(s, _RC)]], v2_ref, sem2)

    def drain_acc(ci, v1_ref, v2_ref):
        s = ci * _RC
        pltpu.make_async_copy(
            o1_hbm.at[idx1_v.at[pl.ds(s, _RC)]], v1_ref, sem1).wait()
        pltpu.make_async_copy(
            o2_hbm.at[idx2_v.at[pl.ds(s, _RC)]], v2_ref, sem2).wait()

        def body(t, a):
            j = t >> 3
            w = (t & 7) << 4
            d = v1_ref[j, pl.ds(w, _L)] - v2_ref[j, pl.ds(w, _L)]
            return a + d * d

        acc = lax.fori_loop(0, _RC * 8, body,
                            jnp.zeros((_L,), jnp.float32), unroll=8)
        acc_v[...] = acc_v[...] + acc

    issue(0, v1a, v2a)

    def pair_body(i, u):
        j = 2 * i
        issue(j + 1, v1b, v2b)
        drain_acc(j, v1a, v2a)
        issue(j + 2, v1a, v2a)
        drain_acc(j + 1, v1b, v2b)
        return u

    lax.fori_loop(0, (_NCH - 2) // 2, pair_body, 0)
    issue(_NCH - 1, v1b, v2b)
    drain_acc(_NCH - 2, v1a, v2a)
    drain_acc(_NCH - 1, v1b, v2b)

    pltpu.sync_copy(acc_v, out_hbm.at[wid])


@jax.jit
def _sc_loss(o1, o2, m1, m2):
    mesh = plsc.VectorSubcoreMesh(core_axis_name="c", subcore_axis_name="s")
    parts = pl.kernel(
        _sc_body,
        out_type=jax.ShapeDtypeStruct((_NW, _L), jnp.float32),
        mesh=mesh,
        compiler_params=pltpu.CompilerParams(needs_layout_passes=False),
        scratch_types=[
            pltpu.VMEM((_PTS,), jnp.int32),
            pltpu.VMEM((_PTS,), jnp.int32),
            pltpu.VMEM((_PTS,), jnp.int32),
            pltpu.VMEM((_PTS * _KC,), jnp.int32),
            pltpu.VMEM((_PTS * _KC,), jnp.int32),
            pltpu.VMEM((_RC, 128), jnp.float32),
            pltpu.VMEM((_RC, 128), jnp.float32),
            pltpu.VMEM((_RC, 128), jnp.float32),
            pltpu.VMEM((_RC, 128), jnp.float32),
            pltpu.VMEM((_L,), jnp.float32),
            pltpu.SemaphoreType.DMA,
            pltpu.SemaphoreType.DMA,
        ],
    )(o1, o2, m1, m2)
    return jnp.sum(parts) * (1.0 / (_B * _N))


def kernel(out_1, out_2, match_1, match_2, nonmatch_2):
    del nonmatch_2

    def rowview(x):
        y = x.transpose(0, 2, 3, 1)
        y = y.reshape(_B, _H, _W // 8, 8, _KC, 128)
        y = y.transpose(0, 1, 2, 4, 3, 5)
        return y.reshape(_V, 128)

    m1 = (match_1[:, :, 0] << 16) | match_1[:, :, 1]
    m2 = (match_2[:, :, 0] << 16) | match_2[:, :, 1]
    return _sc_loss(rowview(out_1), rowview(out_2), m1, m2)
